# baseline (speedup 1.0000x reference)
import functools
import math

import jax
import jax.numpy as jnp
from jax import lax
from jax.experimental import pallas as pl
from jax.experimental.pallas import tpu as pltpu
from jax.experimental.pallas import tpu_sc as plsc

_VOCAB = 1000000
_D = 32
_BATCH = 4096
_SEQ = 200
_SCALE = math.sqrt(_D)

_NC = 2
_NS = 16
_NW = _NC * _NS
_N = _BATCH * _SEQ

_mesh = plsc.VectorSubcoreMesh(core_axis_name="c", subcore_axis_name="s")

_AU = 512
_AMAIN = (_VOCAB // 128) * 128
_AUN = _AMAIN // _AU
_ATAIL = _VOCAB - _AMAIN
_AROWS = _AU * _D // 128
_ATROWS = _ATAIL * _D // 128


@functools.partial(
    pl.kernel,
    mesh=_mesh,
    out_type=jax.ShapeDtypeStruct((_VOCAB * _D // 128, 128), jnp.float32),
    scratch_types=[
        pltpu.VMEM((_D, _AU), jnp.float32),
        pltpu.VMEM((_AROWS, 128), jnp.float32),
        pltpu.VMEM((_D, _ATAIL), jnp.float32),
    ],
    compiler_params=pltpu.CompilerParams(needs_layout_passes=False),
)
def _detranspose(tabt_hbm, tail_hbm, out_hbm, in_v, out_v, tail_v):
    wid = lax.axis_index("s") * _NC + lax.axis_index("c")
    iota = lax.iota(jnp.int32, 16)
    rows_lo = iota
    rows_hi = iota + 16

    def transpose_rows(src_v, nrows):
        def row_body(r, carry):
            base = jnp.full((16,), r * 4, jnp.int32)
            for q in range(8):
                colv = base + (q // 2)
                rv = rows_lo if q % 2 == 0 else rows_hi
                vals = plsc.load_gather(src_v, [rv, colv])
                out_v[r, pl.ds(16 * q, 16)] = vals
            return carry

        lax.fori_loop(0, nrows, row_body, 0)

    nu = (_AUN - wid + _NW - 1) // _NW

    def unit_body(k, carry):
        u = wid + k * _NW
        pltpu.sync_copy(tabt_hbm.at[:, pl.ds(u * _AU, _AU)], in_v)
        transpose_rows(in_v, _AROWS)
        pltpu.sync_copy(out_v, out_hbm.at[pl.ds(u * _AROWS, _AROWS), :])
        return carry

    lax.fori_loop(0, nu, unit_body, 0)

    @pl.when(wid == _NW - 1)
    def _():
        pltpu.sync_copy(tail_hbm, tail_v)
        transpose_rows(tail_v, _ATROWS)
        pltpu.sync_copy(
            out_v.at[pl.ds(0, _ATROWS), :],
            out_hbm.at[pl.ds(_AUN * _AROWS, _ATROWS), :],
        )


_BC = 1024
_BPW = _N // _NW
_BNCH = _BPW // _BC


@functools.partial(
    pl.kernel,
    mesh=_mesh,
    out_type=jax.ShapeDtypeStruct((_N, _D), jnp.float32),
    scratch_types=[
        pltpu.VMEM((_BC,), jnp.int32),
        pltpu.VMEM((_BC, _D), jnp.float32),
        pltpu.SemaphoreType.DMA,
    ],
    compiler_params=pltpu.CompilerParams(use_tc_tiling_on_sc=False),
)
def _gather(x_hbm, tab_hbm, out_hbm, idx_v, rows_v, sem):
    wid = lax.axis_index("s") * _NC + lax.axis_index("c")
    base = wid * _BPW

    def chunk_body(ci, carry):
        off = base + ci * _BC
        pltpu.sync_copy(x_hbm.at[pl.ds(off, _BC)], idx_v)
        pltpu.async_copy(tab_hbm.at[idx_v], rows_v, sem).wait()
        pltpu.sync_copy(rows_v, out_hbm.at[pl.ds(off, _BC)])
        return carry

    lax.fori_loop(0, _BNCH, chunk_body, 0)


@functools.partial(
    pl.kernel,
    mesh=_mesh,
    out_type=jax.ShapeDtypeStruct((_SEQ, _D, _BATCH), jnp.float32),
    scratch_types=[
        pltpu.VMEM((_D, 128), jnp.float32),
        pltpu.VMEM((_D, 128), jnp.float32),
    ],
    compiler_params=pltpu.CompilerParams(needs_layout_passes=False),
)
def _retile(f2_hbm, out_hbm, in_v, out_v):
    wid = lax.axis_index("s") * _NC + lax.axis_index("c")
    iota = lax.iota(jnp.int32, 16)
    rows_q = []
    colbase_q = []
    for q in range(8):
        bl = iota + 16 * q
        rows_q.append(lax.shift_right_logical(bl, 2))
        colbase_q.append((bl & 3) * _D)

    def s_body(s, carry):
        r0 = s * (_BATCH * _D // 128) + wid * _D
        pltpu.sync_copy(f2_hbm.at[pl.ds(r0, _D), :], in_v)

        def d_body(d, carry2):
            for q in range(8):
                vals = plsc.load_gather(in_v, [rows_q[q], colbase_q[q] + d])
                out_v[d, pl.ds(16 * q, 16)] = vals * _SCALE
            return carry2

        lax.fori_loop(0, _D, d_body, 0)
        pltpu.sync_copy(out_v, out_hbm.at[s, :, pl.ds(wid * 128, 128)])
        return carry

    lax.fori_loop(0, _SEQ, s_body, 0)


def kernel(x, emb_table):
    xs = jnp.swapaxes(x, 0, 1).reshape(_N)
    tabt = jnp.swapaxes(emb_table, 0, 1)
    tail = tabt[:, _AMAIN:]
    tab_img = _detranspose(tabt, tail)
    tab2d = tab_img.reshape(_VOCAB, _D)
    rows = _gather(xs, tab2d)
    ot = _retile(rows.reshape(_N * _D // 128, 128))
    return jnp.transpose(ot, (2, 0, 1))


# speedup vs baseline: 1.5546x; 1.5546x over previous
import functools
import math

import jax
import jax.numpy as jnp
from jax import lax
from jax.experimental import pallas as pl
from jax.experimental.pallas import tpu as pltpu
from jax.experimental.pallas import tpu_sc as plsc

_VOCAB = 1000000
_D = 32
_BATCH = 4096
_SEQ = 200
_SCALE = math.sqrt(_D)

_NC = 2
_NS = 16
_NW = _NC * _NS
_N = _BATCH * _SEQ

_mesh = plsc.VectorSubcoreMesh(core_axis_name="c", subcore_axis_name="s")

_AU = 512
_AMAIN = (_VOCAB // 128) * 128
_AUN = _AMAIN // _AU
_ATAIL = _VOCAB - _AMAIN
_AROWS = _AU * _D // 128
_ATROWS = _ATAIL * _D // 128


@functools.partial(
    pl.kernel,
    mesh=_mesh,
    out_type=jax.ShapeDtypeStruct((_VOCAB * _D // 128, 128), jnp.float32),
    scratch_types=[
        pltpu.VMEM((_D, _AU), jnp.float32),
        pltpu.VMEM((_AROWS, 128), jnp.float32),
        pltpu.VMEM((_D, _ATAIL), jnp.float32),
    ],
    compiler_params=pltpu.CompilerParams(needs_layout_passes=False),
)
def _detranspose(tabt_hbm, tail_hbm, out_hbm, in_v, out_v, tail_v):
    wid = lax.axis_index("s") * _NC + lax.axis_index("c")
    iota = lax.iota(jnp.int32, 16)
    rows_lo = iota
    rows_hi = iota + 16

    def transpose_rows(src_v, nrows):
        @plsc.parallel_loop(0, nrows, unroll=4)
        def _(r):
            base = jnp.full((16,), r * 4, jnp.int32)
            for q in range(8):
                colv = base + (q // 2)
                rv = rows_lo if q % 2 == 0 else rows_hi
                vals = plsc.load_gather(src_v, [rv, colv])
                out_v[r, pl.ds(16 * q, 16)] = vals

    nu = (_AUN - wid + _NW - 1) // _NW

    def unit_body(k, carry):
        u = wid + k * _NW
        pltpu.sync_copy(tabt_hbm.at[:, pl.ds(u * _AU, _AU)], in_v)
        transpose_rows(in_v, _AROWS)
        pltpu.sync_copy(out_v, out_hbm.at[pl.ds(u * _AROWS, _AROWS), :])
        return carry

    lax.fori_loop(0, nu, unit_body, 0)

    @pl.when(wid == _NW - 1)
    def _():
        pltpu.sync_copy(tail_hbm, tail_v)
        transpose_rows(tail_v, _ATROWS)
        pltpu.sync_copy(
            out_v.at[pl.ds(0, _ATROWS), :],
            out_hbm.at[pl.ds(_AUN * _AROWS, _ATROWS), :],
        )


_BC = 1024
_BPW = _N // _NW
_BNCH = _BPW // _BC


@functools.partial(
    pl.kernel,
    mesh=_mesh,
    out_type=jax.ShapeDtypeStruct((_N, _D), jnp.float32),
    scratch_types=[
        pltpu.VMEM((_BC,), jnp.int32),
        pltpu.VMEM((_BC, _D), jnp.float32),
        pltpu.SemaphoreType.DMA,
    ],
    compiler_params=pltpu.CompilerParams(use_tc_tiling_on_sc=False),
)
def _gather(x_hbm, tab_hbm, out_hbm, idx_v, rows_v, sem):
    wid = lax.axis_index("s") * _NC + lax.axis_index("c")
    base = wid * _BPW

    def chunk_body(ci, carry):
        off = base + ci * _BC
        pltpu.sync_copy(x_hbm.at[pl.ds(off, _BC)], idx_v)
        pltpu.async_copy(tab_hbm.at[idx_v], rows_v, sem).wait()
        pltpu.sync_copy(rows_v, out_hbm.at[pl.ds(off, _BC)])
        return carry

    lax.fori_loop(0, _BNCH, chunk_body, 0)


@functools.partial(
    pl.kernel,
    mesh=_mesh,
    out_type=jax.ShapeDtypeStruct((_SEQ, _D, _BATCH), jnp.float32),
    scratch_types=[
        pltpu.VMEM((_D, 128), jnp.float32),
        pltpu.VMEM((_D, 128), jnp.float32),
    ],
    compiler_params=pltpu.CompilerParams(needs_layout_passes=False),
)
def _retile(f2_hbm, out_hbm, in_v, out_v):
    wid = lax.axis_index("s") * _NC + lax.axis_index("c")
    iota = lax.iota(jnp.int32, 16)
    rows_q = []
    colbase_q = []
    for q in range(8):
        bl = iota + 16 * q
        rows_q.append(lax.shift_right_logical(bl, 2))
        colbase_q.append((bl & 3) * _D)

    def s_body(s, carry):
        r0 = s * (_BATCH * _D // 128) + wid * _D
        pltpu.sync_copy(f2_hbm.at[pl.ds(r0, _D), :], in_v)

        @plsc.parallel_loop(0, _D, unroll=4)
        def _(d):
            for q in range(8):
                vals = plsc.load_gather(in_v, [rows_q[q], colbase_q[q] + d])
                out_v[d, pl.ds(16 * q, 16)] = vals * _SCALE
        pltpu.sync_copy(out_v, out_hbm.at[s, :, pl.ds(wid * 128, 128)])
        return carry

    lax.fori_loop(0, _SEQ, s_body, 0)


def kernel(x, emb_table):
    xs = jnp.swapaxes(x, 0, 1).reshape(_N)
    tabt = jnp.swapaxes(emb_table, 0, 1)
    tail = tabt[:, _AMAIN:]
    tab_img = _detranspose(tabt, tail)
    tab2d = tab_img.reshape(_VOCAB, _D)
    rows = _gather(xs, tab2d)
    ot = _retile(rows.reshape(_N * _D // 128, 128))
    return jnp.transpose(ot, (2, 0, 1))


# speedup vs baseline: 1.5707x; 1.0104x over previous
import functools
import math

import jax
import jax.numpy as jnp
from jax import lax
from jax.experimental import pallas as pl
from jax.experimental.pallas import tpu as pltpu
from jax.experimental.pallas import tpu_sc as plsc

_VOCAB = 1000000
_D = 32
_BATCH = 4096
_SEQ = 200
_SCALE = math.sqrt(_D)

_NC = 2
_NS = 16
_NW = _NC * _NS
_N = _BATCH * _SEQ

_mesh = plsc.VectorSubcoreMesh(core_axis_name="c", subcore_axis_name="s")

_AU = 512
_AMAIN = (_VOCAB // 128) * 128
_AUN = _AMAIN // _AU
_ATAIL = _VOCAB - _AMAIN
_AROWS = _AU * _D // 128
_ATROWS = _ATAIL * _D // 128


@functools.partial(
    pl.kernel,
    mesh=_mesh,
    out_type=jax.ShapeDtypeStruct((_VOCAB * _D // 128, 128), jnp.float32),
    scratch_types=[
        pltpu.VMEM((_D, _AU), jnp.float32),
        pltpu.VMEM((_AROWS, 128), jnp.float32),
        pltpu.VMEM((_D, _ATAIL), jnp.float32),
    ],
    compiler_params=pltpu.CompilerParams(needs_layout_passes=False),
)
def _detranspose(tabt_hbm, tail_hbm, out_hbm, in_v, out_v, tail_v):
    wid = lax.axis_index("s") * _NC + lax.axis_index("c")
    iota = lax.iota(jnp.int32, 16)
    rows_lo = iota
    rows_hi = iota + 16

    def transpose_rows(src_v, nrows):
        @plsc.parallel_loop(0, nrows, unroll=8)
        def _(r):
            base = jnp.full((16,), r * 4, jnp.int32)
            for q in range(8):
                colv = base + (q // 2)
                rv = rows_lo if q % 2 == 0 else rows_hi
                vals = plsc.load_gather(src_v, [rv, colv])
                out_v[r, pl.ds(16 * q, 16)] = vals

    nu = (_AUN - wid + _NW - 1) // _NW

    def unit_body(k, carry):
        u = wid + k * _NW
        pltpu.sync_copy(tabt_hbm.at[:, pl.ds(u * _AU, _AU)], in_v)
        transpose_rows(in_v, _AROWS)
        pltpu.sync_copy(out_v, out_hbm.at[pl.ds(u * _AROWS, _AROWS), :])
        return carry

    lax.fori_loop(0, nu, unit_body, 0)

    @pl.when(wid == _NW - 1)
    def _():
        pltpu.sync_copy(tail_hbm, tail_v)
        transpose_rows(tail_v, _ATROWS)
        pltpu.sync_copy(
            out_v.at[pl.ds(0, _ATROWS), :],
            out_hbm.at[pl.ds(_AUN * _AROWS, _ATROWS), :],
        )


_BC = 1024
_BPW = _N // _NW
_BNCH = _BPW // _BC


@functools.partial(
    pl.kernel,
    mesh=_mesh,
    out_type=jax.ShapeDtypeStruct((_N, _D), jnp.float32),
    scratch_types=[
        pltpu.VMEM((_BC,), jnp.int32),
        pltpu.VMEM((_BC, _D), jnp.float32),
        pltpu.SemaphoreType.DMA,
    ],
    compiler_params=pltpu.CompilerParams(use_tc_tiling_on_sc=False),
)
def _gather(x_hbm, tab_hbm, out_hbm, idx_v, rows_v, sem):
    wid = lax.axis_index("s") * _NC + lax.axis_index("c")
    base = wid * _BPW

    def chunk_body(ci, carry):
        off = base + ci * _BC
        pltpu.sync_copy(x_hbm.at[pl.ds(off, _BC)], idx_v)
        pltpu.async_copy(tab_hbm.at[idx_v], rows_v, sem).wait()
        pltpu.sync_copy(rows_v, out_hbm.at[pl.ds(off, _BC)])
        return carry

    lax.fori_loop(0, _BNCH, chunk_body, 0)


@functools.partial(
    pl.kernel,
    mesh=_mesh,
    out_type=jax.ShapeDtypeStruct((_SEQ, _D, _BATCH), jnp.float32),
    scratch_types=[
        pltpu.VMEM((_D, 128), jnp.float32),
        pltpu.VMEM((_D, 128), jnp.float32),
    ],
    compiler_params=pltpu.CompilerParams(needs_layout_passes=False),
)
def _retile(f2_hbm, out_hbm, in_v, out_v):
    wid = lax.axis_index("s") * _NC + lax.axis_index("c")
    iota = lax.iota(jnp.int32, 16)
    rows_q = []
    colbase_q = []
    for q in range(8):
        bl = iota + 16 * q
        rows_q.append(lax.shift_right_logical(bl, 2))
        colbase_q.append((bl & 3) * _D)

    def s_body(s, carry):
        r0 = s * (_BATCH * _D // 128) + wid * _D
        pltpu.sync_copy(f2_hbm.at[pl.ds(r0, _D), :], in_v)

        @plsc.parallel_loop(0, _D, unroll=8)
        def _(d):
            for q in range(8):
                vals = plsc.load_gather(in_v, [rows_q[q], colbase_q[q] + d])
                out_v[d, pl.ds(16 * q, 16)] = vals * _SCALE
        pltpu.sync_copy(out_v, out_hbm.at[s, :, pl.ds(wid * 128, 128)])
        return carry

    lax.fori_loop(0, _SEQ, s_body, 0)


def kernel(x, emb_table):
    xs = jnp.swapaxes(x, 0, 1).reshape(_N)
    tabt = jnp.swapaxes(emb_table, 0, 1)
    tail = tabt[:, _AMAIN:]
    tab_img = _detranspose(tabt, tail)
    tab2d = tab_img.reshape(_VOCAB, _D)
    rows = _gather(xs, tab2d)
    ot = _retile(rows.reshape(_N * _D // 128, 128))
    return jnp.transpose(ot, (2, 0, 1))


# speedup vs baseline: 1.8214x; 1.1596x over previous
import functools
import math

import jax
import jax.numpy as jnp
from jax import lax
from jax.experimental import pallas as pl
from jax.experimental.pallas import tpu as pltpu
from jax.experimental.pallas import tpu_sc as plsc

_VOCAB = 1000000
_D = 32
_BATCH = 4096
_SEQ = 200
_SCALE = math.sqrt(_D)

_NC = 2
_NS = 16
_NW = _NC * _NS

_N = _BATCH * _SEQ
_NPW = _N // _NW
_C = 1024
_NCHUNK = _NPW // _C
_UNROLL = 8

_mesh = plsc.VectorSubcoreMesh(core_axis_name="c", subcore_axis_name="s")


@functools.partial(
    pl.kernel,
    mesh=_mesh,
    out_type=jax.ShapeDtypeStruct((_N, _D), jnp.float32),
    scratch_types=[
        pltpu.VMEM((_C,), jnp.int32),
        pltpu.VMEM((_C, _D), jnp.float32),
        pltpu.SemaphoreType.DMA,
    ],
    compiler_params=pltpu.CompilerParams(use_tc_tiling_on_sc=False),
)
def _emb_lookup(x_hbm, tab_hbm, out_hbm, idx_v, rows_v, sem):
    wid = lax.axis_index("s") * _NC + lax.axis_index("c")
    base = wid * _NPW

    def chunk_body(ci, carry):
        off = base + ci * _C
        pltpu.sync_copy(x_hbm.at[pl.ds(off, _C)], idx_v)
        pltpu.async_copy(tab_hbm.at[idx_v], rows_v, sem).wait()

        def scale_body(i, carry2):
            r0 = i * _UNROLL
            for u in range(_UNROLL):
                for h in range(_D // 16):
                    sl = (r0 + u, pl.ds(h * 16, 16))
                    rows_v[sl] = rows_v[sl] * _SCALE
            return carry2

        lax.fori_loop(0, _C // _UNROLL, scale_body, 0)
        pltpu.sync_copy(rows_v, out_hbm.at[pl.ds(off, _C)])
        return carry

    lax.fori_loop(0, _NCHUNK, chunk_body, 0)


def kernel(x, emb_table):
    out = _emb_lookup(x.reshape(_N), emb_table)
    return out.reshape(_BATCH, _SEQ, _D)
